# Initial kernel scaffold; baseline (speedup 1.0000x reference)
import jax
import jax.numpy as jnp
from jax.experimental import pallas as pl


def kernel(x, fc1_w, star_scale, star_bias, conv_w, fc2_w):
    raise NotImplementedError("write your pallas kernel here")


# speedup vs baseline: 21.1623x; 21.1623x over previous
import jax
import jax.numpy as jnp
from jax.experimental import pallas as pl

K = 9
SUB = 16


_HI = jax.lax.Precision.HIGHEST


def _fold_kernel(fc2_ref, cw_ref, wf_ref):
    fc2 = fc2_ref[...]
    for k in range(K):
        wf_ref[k] = jnp.dot(fc2, cw_ref[k], precision=_HI,
                            preferred_element_type=jnp.float32)


def _main_kernel(x_ref, fc1_ref, ss_ref, sb_ref, wf_ref, out_ref):
    x = x_ref[0]
    fc1 = fc1_ref[...]
    s = ss_ref[0, 0]
    b = sb_ref[0, 0]
    C2 = fc1.shape[0]
    N = x.shape[1]
    M = N // SUB

    h = jnp.dot(fc1.astype(jnp.bfloat16), x.astype(jnp.bfloat16),
                preferred_element_type=jnp.float32)
    h = s * jnp.square(jnp.maximum(h, 0.0)) + b

    row = jax.lax.broadcasted_iota(jnp.int32, (N, M), 0)
    col = jax.lax.broadcasted_iota(jnp.int32, (N, M), 1)
    sel_nm = (row == col * SUB).astype(jnp.float32)
    cand = jnp.dot(h, sel_nm, precision=_HI,
                   preferred_element_type=jnp.float32)

    n2 = jnp.sum(h * h, axis=0, keepdims=True)
    csq = cand * cand
    ones = jnp.ones((C2, 1), jnp.float32)
    c2 = jax.lax.dot_general(csq, ones, (((0,), (0,)), ((), ())),
                             precision=_HI,
                             preferred_element_type=jnp.float32)
    d = jax.lax.dot_general(cand.astype(jnp.bfloat16), h.astype(jnp.bfloat16),
                            (((0,), (0,)), ((), ())),
                            preferred_element_type=jnp.float32)
    neg = 2.0 * d - c2 - n2

    iota = jax.lax.broadcasted_iota(jnp.int32, (M, N), 0)
    acc = jnp.zeros((out_ref.shape[1], N), jnp.float32)
    for k in range(K):
        mx = jnp.max(neg, axis=0, keepdims=True)
        ismax = neg >= mx
        sel = jnp.min(jnp.where(ismax, iota, M), axis=0, keepdims=True)
        onehot = iota == sel
        vk = jnp.dot(wf_ref[k], cand, precision=_HI,
                     preferred_element_type=jnp.float32)
        acc = acc + jnp.dot(vk, onehot.astype(jnp.float32), precision=_HI,
                            preferred_element_type=jnp.float32)
        neg = jnp.where(onehot, -jnp.inf, neg)
    out_ref[0] = acc


def kernel(x, fc1_w, star_scale, star_bias, conv_w, fc2_w):
    B, C, H, W = x.shape
    N = H * W
    C2 = fc1_w.shape[0]
    xf = x.reshape(B, C, N)
    cw = jnp.transpose(conv_w, (2, 0, 1))

    wf = pl.pallas_call(
        _fold_kernel,
        out_shape=jax.ShapeDtypeStruct((K, C, C2), jnp.float32),
    )(fc2_w, cw)

    ss = jnp.reshape(star_scale, (1, 1)).astype(jnp.float32)
    sb = jnp.reshape(star_bias, (1, 1)).astype(jnp.float32)

    out = pl.pallas_call(
        _main_kernel,
        grid=(B,),
        in_specs=[
            pl.BlockSpec((1, C, N), lambda i: (i, 0, 0)),
            pl.BlockSpec((C2, C), lambda i: (0, 0)),
            pl.BlockSpec((1, 1), lambda i: (0, 0)),
            pl.BlockSpec((1, 1), lambda i: (0, 0)),
            pl.BlockSpec((K, C, C2), lambda i: (0, 0, 0)),
        ],
        out_specs=pl.BlockSpec((1, C, N), lambda i: (i, 0, 0)),
        out_shape=jax.ShapeDtypeStruct((B, C, N), jnp.float32),
    )(xf, fc1_w, ss, sb, wf)
    return out.reshape(B, C, H, W)
